# Initial kernel scaffold; baseline (speedup 1.0000x reference)
import functools

import jax
import jax.numpy as jnp
from jax import lax
from jax.experimental import pallas as pl
from jax.experimental.pallas import tpu as pltpu
from jax.experimental.pallas import tpu_sc as plsc

N = 10000
D = 128
E = 320000

NC = 2
NS = 16
NW = NC * NS
EPW = E // NW
CH = 80
NCH = EPW // CH
NPT = N // NS
DEGW = 16

_mesh = plsc.VectorSubcoreMesh(
    core_axis_name="c", subcore_axis_name="s", num_cores=NC, num_subcores=NS
)


@functools.partial(
    pl.kernel,
    out_type=jax.ShapeDtypeStruct((NC, N, DEGW), jnp.float32),
    mesh=_mesh,
    scratch_types=[
        pltpu.VMEM((CH,), jnp.int32),
        pltpu.VMEM((CH, DEGW), jnp.float32),
        pltpu.VMEM_SHARED((N, DEGW), jnp.float32),
        pltpu.SemaphoreType.DMA,
    ],
)
def _sc_degree(dst_hbm, ones_hbm, zeros_hbm, out_hbm, idx_v, ones_v, acc_sh, sem):
    cid = lax.axis_index("c")
    sid = lax.axis_index("s")
    wid = sid * NC + cid
    pltpu.sync_copy(ones_hbm, ones_v)
    pltpu.sync_copy(zeros_hbm, acc_sh.at[pl.ds(sid * NPT, NPT)])
    plsc.subcore_barrier()

    def body(i, carry):
        base = wid * EPW + i * CH
        pltpu.sync_copy(dst_hbm.at[pl.ds(base, CH)], idx_v)
        pltpu.sync_copy(ones_v, acc_sh.at[idx_v], add=True)
        return carry

    lax.fori_loop(0, NCH, body, 0)
    plsc.subcore_barrier()
    pltpu.sync_copy(
        acc_sh.at[pl.ds(sid * NPT, NPT)],
        out_hbm.at[cid, pl.ds(sid * NPT, NPT)],
    )


@functools.partial(
    pl.kernel,
    out_type=jax.ShapeDtypeStruct((NC, N, D), jnp.float32),
    mesh=_mesh,
    scratch_types=[
        pltpu.VMEM((CH,), jnp.int32),
        pltpu.VMEM((CH,), jnp.int32),
        pltpu.VMEM((CH, D), jnp.float32),
        pltpu.VMEM_SHARED((N, D), jnp.float32),
        pltpu.SemaphoreType.DMA,
    ],
)
def _sc_scatter(y_hbm, src_hbm, dst_hbm, zeros_hbm, out_hbm,
                sidx_v, didx_v, rows_v, acc_sh, sem):
    cid = lax.axis_index("c")
    sid = lax.axis_index("s")
    wid = sid * NC + cid
    pltpu.sync_copy(zeros_hbm, acc_sh.at[pl.ds(sid * NPT, NPT)])
    plsc.subcore_barrier()

    def body(i, carry):
        base = wid * EPW + i * CH
        pltpu.sync_copy(src_hbm.at[pl.ds(base, CH)], sidx_v)
        pltpu.sync_copy(dst_hbm.at[pl.ds(base, CH)], didx_v)
        pltpu.async_copy(y_hbm.at[sidx_v], rows_v, sem).wait()
        pltpu.sync_copy(rows_v, acc_sh.at[didx_v], add=True)
        return carry

    lax.fori_loop(0, NCH, body, 0)
    plsc.subcore_barrier()
    pltpu.sync_copy(
        acc_sh.at[pl.ds(sid * NPT, NPT)],
        out_hbm.at[cid, pl.ds(sid * NPT, NPT)],
    )


_BLK = 1000


def _linear_body(deg_ref, x_ref, w_ref, y_ref):
    deg = deg_ref[0, :, 0:1] + deg_ref[1, :, 0:1] + 1.0
    dinv = lax.rsqrt(deg)
    xw = jnp.dot(x_ref[...], w_ref[...], preferred_element_type=jnp.float32)
    y_ref[...] = xw * dinv


def _finish_body(deg_ref, acc_ref, y_ref, b_ref, o_ref):
    deg = deg_ref[0, :, 0:1] + deg_ref[1, :, 0:1] + 1.0
    dinv = lax.rsqrt(deg)
    s = acc_ref[0] + acc_ref[1] + y_ref[...]
    o_ref[...] = jnp.maximum(s * dinv + b_ref[...], 0.0)


def kernel(x, edge_index, W, b):
    src = edge_index[0]
    dst = edge_index[1]
    ones16 = jnp.ones((CH, DEGW), jnp.float32)
    zeros16 = jnp.zeros((NPT, DEGW), jnp.float32)
    zeros128 = jnp.zeros((NPT, D), jnp.float32)

    deg16 = _sc_degree(dst, ones16, zeros16)

    y = pl.pallas_call(
        _linear_body,
        grid=(N // _BLK,),
        in_specs=[
            pl.BlockSpec((NC, _BLK, DEGW), lambda i: (0, i, 0)),
            pl.BlockSpec((_BLK, D), lambda i: (i, 0)),
            pl.BlockSpec((D, D), lambda i: (0, 0)),
        ],
        out_specs=pl.BlockSpec((_BLK, D), lambda i: (i, 0)),
        out_shape=jax.ShapeDtypeStruct((N, D), jnp.float32),
    )(deg16, x, W)

    acc = _sc_scatter(y, src, dst, zeros128)

    out = pl.pallas_call(
        _finish_body,
        grid=(N // _BLK,),
        in_specs=[
            pl.BlockSpec((NC, _BLK, DEGW), lambda i: (0, i, 0)),
            pl.BlockSpec((NC, _BLK, D), lambda i: (0, i, 0)),
            pl.BlockSpec((_BLK, D), lambda i: (i, 0)),
            pl.BlockSpec((1, D), lambda i: (0, 0)),
        ],
        out_specs=pl.BlockSpec((_BLK, D), lambda i: (i, 0)),
        out_shape=jax.ShapeDtypeStruct((N, D), jnp.float32),
    )(deg16, acc, y, b.reshape(1, D))
    return out


# speedup vs baseline: 15.8977x; 15.8977x over previous
import functools

import jax
import jax.numpy as jnp
from jax import lax
from jax.experimental import pallas as pl
from jax.experimental.pallas import tpu as pltpu
from jax.experimental.pallas import tpu_sc as plsc

N = 10000
D = 128
E = 320000

NC = 2
NS = 16
NW = NC * NS
EPW = E // NW
CH = 80
NCH = EPW // CH
NPAD = 10240
NPT = NPAD // NS
DEGW = 128

_mesh = plsc.VectorSubcoreMesh(
    core_axis_name="c", subcore_axis_name="s", num_cores=NC, num_subcores=NS
)


def _make_sc_degree(degw):
    @functools.partial(
        pl.kernel,
        out_type=jax.ShapeDtypeStruct((NC, NPAD, degw), jnp.float32),
        mesh=_mesh,
        scratch_types=[
            pltpu.VMEM((CH,), jnp.int32),
            pltpu.VMEM((CH, degw), jnp.float32),
            pltpu.VMEM_SHARED((NPAD, degw), jnp.float32),
            pltpu.SemaphoreType.DMA,
        ],
    )
    def sc_degree(dst_hbm, ones_hbm, zeros_hbm, out_hbm, idx_v, ones_v,
                  acc_sh, sem):
        cid = lax.axis_index("c")
        sid = lax.axis_index("s")
        wid = sid * NC + cid
        pltpu.sync_copy(ones_hbm, ones_v)
        pltpu.sync_copy(zeros_hbm, acc_sh.at[pl.ds(sid * NPT, NPT)])
        plsc.subcore_barrier()

        def body(i, carry):
            base = wid * EPW + i * CH
            pltpu.sync_copy(dst_hbm.at[pl.ds(base, CH)], idx_v)
            pltpu.sync_copy(ones_v, acc_sh.at[idx_v], add=True)
            return carry

        lax.fori_loop(0, NCH, body, 0)
        plsc.subcore_barrier()
        pltpu.sync_copy(
            acc_sh.at[pl.ds(sid * NPT, NPT)],
            out_hbm.at[cid, pl.ds(sid * NPT, NPT)],
        )

    return sc_degree


_sc_degree = _make_sc_degree(DEGW)


@functools.partial(
    pl.kernel,
    out_type=jax.ShapeDtypeStruct((NC, NPAD, D), jnp.float32),
    mesh=_mesh,
    scratch_types=[
        pltpu.VMEM((CH,), jnp.int32),
        pltpu.VMEM((CH,), jnp.int32),
        pltpu.VMEM((CH, D), jnp.float32),
        pltpu.VMEM_SHARED((NPAD, D), jnp.float32),
        pltpu.SemaphoreType.DMA,
    ],
)
def _sc_scatter(y_hbm, src_hbm, dst_hbm, zeros_hbm, out_hbm,
                sidx_v, didx_v, rows_v, acc_sh, sem):
    cid = lax.axis_index("c")
    sid = lax.axis_index("s")
    wid = sid * NC + cid
    pltpu.sync_copy(zeros_hbm, acc_sh.at[pl.ds(sid * NPT, NPT)])
    plsc.subcore_barrier()

    def body(i, carry):
        base = wid * EPW + i * CH
        pltpu.sync_copy(src_hbm.at[pl.ds(base, CH)], sidx_v)
        pltpu.sync_copy(dst_hbm.at[pl.ds(base, CH)], didx_v)
        pltpu.async_copy(y_hbm.at[sidx_v], rows_v, sem).wait()
        pltpu.sync_copy(rows_v, acc_sh.at[didx_v], add=True)
        return carry

    lax.fori_loop(0, NCH, body, 0)
    plsc.subcore_barrier()
    pltpu.sync_copy(
        acc_sh.at[pl.ds(sid * NPT, NPT)],
        out_hbm.at[cid, pl.ds(sid * NPT, NPT)],
    )


_BLK = 1000


def _linear_body(deg_ref, x_ref, w_ref, y_ref):
    deg = deg_ref[0] + deg_ref[1] + 1.0
    dinv = lax.rsqrt(deg)
    xw = jnp.dot(x_ref[...], w_ref[...], preferred_element_type=jnp.float32)
    y_ref[...] = xw * dinv


def _finish_body(deg_ref, acc_ref, y_ref, b_ref, o_ref):
    deg = deg_ref[0] + deg_ref[1] + 1.0
    dinv = lax.rsqrt(deg)
    s = acc_ref[0] + acc_ref[1] + y_ref[...]
    o_ref[...] = jnp.maximum(s * dinv + b_ref[...], 0.0)


def kernel(x, edge_index, W, b):
    src = edge_index[0]
    dst = edge_index[1]
    ones16 = jnp.ones((CH, DEGW), jnp.float32)
    zeros16 = jnp.zeros((NPT, DEGW), jnp.float32)
    zeros128 = jnp.zeros((NPT, D), jnp.float32)

    deg16 = _sc_degree(dst, ones16, zeros16)

    y = pl.pallas_call(
        _linear_body,
        grid=(N // _BLK,),
        in_specs=[
            pl.BlockSpec((NC, _BLK, DEGW), lambda i: (0, i, 0)),
            pl.BlockSpec((_BLK, D), lambda i: (i, 0)),
            pl.BlockSpec((D, D), lambda i: (0, 0)),
        ],
        out_specs=pl.BlockSpec((_BLK, D), lambda i: (i, 0)),
        out_shape=jax.ShapeDtypeStruct((N, D), jnp.float32),
    )(deg16, x, W)

    acc = _sc_scatter(y, src, dst, zeros128)

    out = pl.pallas_call(
        _finish_body,
        grid=(N // _BLK,),
        in_specs=[
            pl.BlockSpec((NC, _BLK, DEGW), lambda i: (0, i, 0)),
            pl.BlockSpec((NC, _BLK, D), lambda i: (0, i, 0)),
            pl.BlockSpec((_BLK, D), lambda i: (i, 0)),
            pl.BlockSpec((1, D), lambda i: (0, 0)),
        ],
        out_specs=pl.BlockSpec((_BLK, D), lambda i: (i, 0)),
        out_shape=jax.ShapeDtypeStruct((N, D), jnp.float32),
    )(deg16, acc, y, b.reshape(1, D))
    return out


# speedup vs baseline: 24.5459x; 1.5440x over previous
import functools

import jax
import jax.numpy as jnp
from jax import lax
from jax.experimental import pallas as pl
from jax.experimental.pallas import tpu as pltpu
from jax.experimental.pallas import tpu_sc as plsc

N = 10000
D = 128
E = 320000

NC = 2
NS = 16
NW = NC * NS
EPW = E // NW
CH = 40
SB = 25
NSB = EPW // (SB * CH)
NPAD = 10240
NPT = NPAD // NS
NBUF = 2

_mesh = plsc.VectorSubcoreMesh(
    core_axis_name="c", subcore_axis_name="s", num_cores=NC, num_subcores=NS
)


@functools.partial(
    pl.kernel,
    out_type=jax.ShapeDtypeStruct((NC, NPAD, D), jnp.float32),
    mesh=_mesh,
    scratch_types=[
        pltpu.VMEM((SB, CH), jnp.int32),
        pltpu.VMEM((CH, D), jnp.float32),
        pltpu.SemaphoreType.DMA((NBUF,)),
        pltpu.VMEM_SHARED((NPAD, D), jnp.float32),
    ],
)
def _sc_degree(dst_hbm, ones_hbm, zeros_hbm, out_hbm, didx_v, ones_v, ssem,
               acc_sh):
    cid = lax.axis_index("c")
    sid = lax.axis_index("s")
    wid = sid * NC + cid
    pltpu.sync_copy(ones_hbm, ones_v)
    pltpu.sync_copy(zeros_hbm, acc_sh.at[pl.ds(sid * NPT, NPT)])
    plsc.subcore_barrier()

    def outer(ob, carry):
        pltpu.sync_copy(dst_hbm.at[wid, ob], didx_v)

        def body(j, carry2):
            descs = [
                pltpu.async_copy(
                    ones_v, acc_sh.at[didx_v.at[j * NBUF + b]], ssem.at[b],
                    add=True)
                for b in range(NBUF)
            ]
            for d in descs:
                d.wait()
            return carry2

        lax.fori_loop(0, SB // NBUF, body, 0)
        for c in range(SB - SB % NBUF, SB):
            pltpu.sync_copy(ones_v, acc_sh.at[didx_v.at[c]], add=True)
        return carry

    lax.fori_loop(0, NSB, outer, 0)
    plsc.subcore_barrier()
    pltpu.sync_copy(
        acc_sh.at[pl.ds(sid * NPT, NPT)],
        out_hbm.at[cid, pl.ds(sid * NPT, NPT)],
    )


@functools.partial(
    pl.kernel,
    out_type=jax.ShapeDtypeStruct((NC, NPAD, D), jnp.float32),
    mesh=_mesh,
    scratch_types=[
        pltpu.VMEM((SB, CH), jnp.int32),
        pltpu.VMEM((SB, CH), jnp.int32),
        pltpu.VMEM((NBUF, CH, D), jnp.float32),
        pltpu.SemaphoreType.DMA((NBUF,)),
        pltpu.SemaphoreType.DMA((NBUF,)),
        pltpu.VMEM_SHARED((NPAD, D), jnp.float32),
    ],
)
def _sc_scatter(y_hbm, src_hbm, dst_hbm, zeros_hbm, out_hbm,
                sidx_v, didx_v, rows_v, gsem, ssem, acc_sh):
    cid = lax.axis_index("c")
    sid = lax.axis_index("s")
    wid = sid * NC + cid
    pltpu.sync_copy(zeros_hbm, acc_sh.at[pl.ds(sid * NPT, NPT)])
    plsc.subcore_barrier()

    def gather(c, b):
        return pltpu.async_copy(y_hbm.at[sidx_v.at[c]], rows_v.at[b],
                                gsem.at[b])

    def outer(ob, carry):
        pltpu.sync_copy(src_hbm.at[wid, ob], sidx_v)
        pltpu.sync_copy(dst_hbm.at[wid, ob], didx_v)
        for b in range(NBUF):
            gather(b, b)

        def body(c, carry2):
            b = lax.rem(c, NBUF)
            for bb in range(NBUF):

                @pl.when(b == bb)
                def _():
                    pltpu.make_async_copy(y_hbm.at[sidx_v.at[c]],
                                          rows_v.at[bb], gsem.at[bb]).wait()
                    pltpu.async_copy(rows_v.at[bb], acc_sh.at[didx_v.at[c]],
                                     ssem.at[bb], add=True)

                    @pl.when(c + NBUF < SB)
                    def _():
                        pltpu.make_async_copy(
                            rows_v.at[bb], acc_sh.at[didx_v.at[c]],
                            ssem.at[bb]).wait()
                        gather(c + NBUF, bb)

            return carry2

        lax.fori_loop(0, SB, body, 0)
        for c in range(SB - NBUF, SB):
            b = c % NBUF
            pltpu.make_async_copy(rows_v.at[b], acc_sh.at[didx_v.at[c]],
                                  ssem.at[b]).wait()
        return carry

    lax.fori_loop(0, NSB, outer, 0)
    plsc.subcore_barrier()
    pltpu.sync_copy(
        acc_sh.at[pl.ds(sid * NPT, NPT)],
        out_hbm.at[cid, pl.ds(sid * NPT, NPT)],
    )


_BLK = 1000


def _linear_body(deg_ref, x_ref, w_ref, y_ref):
    deg = deg_ref[0] + deg_ref[1] + 1.0
    dinv = lax.rsqrt(deg)
    xw = jnp.dot(x_ref[...], w_ref[...], preferred_element_type=jnp.float32)
    y_ref[...] = xw * dinv


def _finish_body(deg_ref, acc_ref, y_ref, b_ref, o_ref):
    deg = deg_ref[0] + deg_ref[1] + 1.0
    dinv = lax.rsqrt(deg)
    s = acc_ref[0] + acc_ref[1] + y_ref[...]
    o_ref[...] = jnp.maximum(s * dinv + b_ref[...], 0.0)


def kernel(x, edge_index, W, b):
    src = edge_index[0].reshape(NW, NSB, SB, CH)
    dst = edge_index[1].reshape(NW, NSB, SB, CH)
    ones_rows = jnp.ones((CH, D), jnp.float32)
    zeros_rows = jnp.zeros((NPT, D), jnp.float32)

    deg = _sc_degree(dst, ones_rows, zeros_rows)

    y = pl.pallas_call(
        _linear_body,
        grid=(N // _BLK,),
        in_specs=[
            pl.BlockSpec((NC, _BLK, D), lambda i: (0, i, 0)),
            pl.BlockSpec((_BLK, D), lambda i: (i, 0)),
            pl.BlockSpec((D, D), lambda i: (0, 0)),
        ],
        out_specs=pl.BlockSpec((_BLK, D), lambda i: (i, 0)),
        out_shape=jax.ShapeDtypeStruct((N, D), jnp.float32),
    )(deg, x, W)

    acc = _sc_scatter(y, src, dst, zeros_rows)

    out = pl.pallas_call(
        _finish_body,
        grid=(N // _BLK,),
        in_specs=[
            pl.BlockSpec((NC, _BLK, D), lambda i: (0, i, 0)),
            pl.BlockSpec((NC, _BLK, D), lambda i: (0, i, 0)),
            pl.BlockSpec((_BLK, D), lambda i: (i, 0)),
            pl.BlockSpec((1, D), lambda i: (0, 0)),
        ],
        out_specs=pl.BlockSpec((_BLK, D), lambda i: (i, 0)),
        out_shape=jax.ShapeDtypeStruct((N, D), jnp.float32),
    )(deg, acc, y, b.reshape(1, D))
    return out


# speedup vs baseline: 32.3138x; 1.3165x over previous
import functools

import jax
import jax.numpy as jnp
from jax import lax
from jax.experimental import pallas as pl
from jax.experimental.pallas import tpu as pltpu
from jax.experimental.pallas import tpu_sc as plsc

N = 10000
D = 128
E = 320000

NC = 2
NS = 16
NW = NC * NS
EPW = E // NW
CH = 80
SB = 25
NSB = EPW // (SB * CH)
NPAD = 10240
NPT = NPAD // NS
NBUF = 4

_mesh = plsc.VectorSubcoreMesh(
    core_axis_name="c", subcore_axis_name="s", num_cores=NC, num_subcores=NS
)


@functools.partial(
    pl.kernel,
    out_type=jax.ShapeDtypeStruct((NC, NPAD, D), jnp.float32),
    mesh=_mesh,
    scratch_types=[
        pltpu.VMEM((SB, CH), jnp.int32),
        pltpu.VMEM((CH, D), jnp.float32),
        pltpu.SemaphoreType.DMA((NBUF,)),
        pltpu.VMEM_SHARED((NPAD, D), jnp.float32),
    ],
)
def _sc_degree(dst_hbm, ones_hbm, zeros_hbm, out_hbm, didx_v, ones_v, ssem,
               acc_sh):
    cid = lax.axis_index("c")
    sid = lax.axis_index("s")
    wid = sid * NC + cid
    pltpu.sync_copy(ones_hbm, ones_v)
    pltpu.sync_copy(zeros_hbm, acc_sh.at[pl.ds(sid * NPT, NPT)])
    plsc.subcore_barrier()

    def outer(ob, carry):
        pltpu.sync_copy(dst_hbm.at[wid, ob], didx_v)

        def body(j, carry2):
            descs = [
                pltpu.async_copy(
                    ones_v, acc_sh.at[didx_v.at[j * NBUF + b]], ssem.at[b],
                    add=True)
                for b in range(NBUF)
            ]
            for d in descs:
                d.wait()
            return carry2

        lax.fori_loop(0, SB // NBUF, body, 0)
        for c in range(SB - SB % NBUF, SB):
            pltpu.sync_copy(ones_v, acc_sh.at[didx_v.at[c]], add=True)
        return carry

    lax.fori_loop(0, NSB, outer, 0)
    plsc.subcore_barrier()
    pltpu.sync_copy(
        acc_sh.at[pl.ds(sid * NPT, NPT)],
        out_hbm.at[cid, pl.ds(sid * NPT, NPT)],
    )


@functools.partial(
    pl.kernel,
    out_type=jax.ShapeDtypeStruct((NC, NPAD, D), jnp.float32),
    mesh=_mesh,
    scratch_types=[
        pltpu.VMEM((SB, CH), jnp.int32),
        pltpu.VMEM((SB, CH), jnp.int32),
        pltpu.VMEM((NBUF, CH, D), jnp.float32),
        pltpu.SemaphoreType.DMA((NBUF,)),
        pltpu.SemaphoreType.DMA((NBUF,)),
        pltpu.VMEM_SHARED((NPAD, D), jnp.float32),
    ],
)
def _sc_scatter(y_hbm, src_hbm, dst_hbm, zeros_hbm, out_hbm,
                sidx_v, didx_v, rows_v, gsem, ssem, acc_sh):
    cid = lax.axis_index("c")
    sid = lax.axis_index("s")
    wid = sid * NC + cid
    pltpu.sync_copy(zeros_hbm, acc_sh.at[pl.ds(sid * NPT, NPT)])
    plsc.subcore_barrier()

    def gather(c, b):
        return pltpu.async_copy(y_hbm.at[sidx_v.at[c]], rows_v.at[b],
                                gsem.at[b])

    def outer(ob, carry):
        pltpu.sync_copy(src_hbm.at[wid, ob], sidx_v)
        pltpu.sync_copy(dst_hbm.at[wid, ob], didx_v)
        for b in range(NBUF):
            gather(b, b)

        def body(c, carry2):
            b = lax.rem(c, NBUF)
            for bb in range(NBUF):

                @pl.when(b == bb)
                def _():
                    pltpu.make_async_copy(y_hbm.at[sidx_v.at[c]],
                                          rows_v.at[bb], gsem.at[bb]).wait()
                    pltpu.async_copy(rows_v.at[bb], acc_sh.at[didx_v.at[c]],
                                     ssem.at[bb], add=True)

                    @pl.when(c + NBUF < SB)
                    def _():
                        pltpu.make_async_copy(
                            rows_v.at[bb], acc_sh.at[didx_v.at[c]],
                            ssem.at[bb]).wait()
                        gather(c + NBUF, bb)

            return carry2

        lax.fori_loop(0, SB, body, 0)
        for c in range(SB - NBUF, SB):
            b = c % NBUF
            pltpu.make_async_copy(rows_v.at[b], acc_sh.at[didx_v.at[c]],
                                  ssem.at[b]).wait()
        return carry

    lax.fori_loop(0, NSB, outer, 0)
    plsc.subcore_barrier()
    pltpu.sync_copy(
        acc_sh.at[pl.ds(sid * NPT, NPT)],
        out_hbm.at[cid, pl.ds(sid * NPT, NPT)],
    )


_BLK = 1000


def _linear_body(deg_ref, x_ref, w_ref, y_ref):
    deg = deg_ref[0] + deg_ref[1] + 1.0
    dinv = lax.rsqrt(deg)
    xw = jnp.dot(x_ref[...], w_ref[...], preferred_element_type=jnp.float32)
    y_ref[...] = xw * dinv


def _finish_body(deg_ref, acc_ref, y_ref, b_ref, o_ref):
    deg = deg_ref[0] + deg_ref[1] + 1.0
    dinv = lax.rsqrt(deg)
    s = acc_ref[0] + acc_ref[1] + y_ref[...]
    o_ref[...] = jnp.maximum(s * dinv + b_ref[...], 0.0)


def kernel(x, edge_index, W, b):
    src = edge_index[0].reshape(NW, NSB, SB, CH)
    dst = edge_index[1].reshape(NW, NSB, SB, CH)
    ones_rows = jnp.ones((CH, D), jnp.float32)
    zeros_rows = jnp.zeros((NPT, D), jnp.float32)

    deg = _sc_degree(dst, ones_rows, zeros_rows)

    y = pl.pallas_call(
        _linear_body,
        grid=(N // _BLK,),
        in_specs=[
            pl.BlockSpec((NC, _BLK, D), lambda i: (0, i, 0)),
            pl.BlockSpec((_BLK, D), lambda i: (i, 0)),
            pl.BlockSpec((D, D), lambda i: (0, 0)),
        ],
        out_specs=pl.BlockSpec((_BLK, D), lambda i: (i, 0)),
        out_shape=jax.ShapeDtypeStruct((N, D), jnp.float32),
    )(deg, x, W)

    acc = _sc_scatter(y, src, dst, zeros_rows)

    out = pl.pallas_call(
        _finish_body,
        grid=(N // _BLK,),
        in_specs=[
            pl.BlockSpec((NC, _BLK, D), lambda i: (0, i, 0)),
            pl.BlockSpec((NC, _BLK, D), lambda i: (0, i, 0)),
            pl.BlockSpec((_BLK, D), lambda i: (i, 0)),
            pl.BlockSpec((1, D), lambda i: (0, 0)),
        ],
        out_specs=pl.BlockSpec((_BLK, D), lambda i: (i, 0)),
        out_shape=jax.ShapeDtypeStruct((N, D), jnp.float32),
    )(deg, acc, y, b.reshape(1, D))
    return out
